# Initial kernel scaffold; baseline (speedup 1.0000x reference)
import jax
import jax.numpy as jnp
from jax.experimental import pallas as pl


def kernel(image):
    raise NotImplementedError("write your pallas kernel here")


# speedup vs baseline: 67.6624x; 67.6624x over previous
import functools

import jax
import jax.numpy as jnp
from jax import lax
from jax.experimental import pallas as pl
from jax.experimental.pallas import tpu as pltpu
from jax.experimental.pallas import tpu_sc as plsc

H = 224
W = 224
T = H * W
C = 3
WPAD = 240
IOFF = 8
NROWS = 7
NF_ROWS = 11
IMG_ROWS = 13
NV = W // 16


def _rsqrt(x):
    i = plsc.bitcast(x, jnp.int32)
    i = jnp.int32(0x5F3759DF) - (i >> 1)
    y = plsc.bitcast(i, jnp.float32)
    for _ in range(3):
        y = y * (1.5 - 0.5 * x * y * y)
    return y


@functools.partial(
    pl.kernel,
    out_type=jax.ShapeDtypeStruct((T,), jnp.float32),
    mesh=plsc.VectorSubcoreMesh(core_axis_name="c", subcore_axis_name="s"),
    compiler_params=pltpu.CompilerParams(needs_layout_passes=False),
    scratch_types=[
        pltpu.VMEM((C * IMG_ROWS * WPAD,), jnp.float32),
        pltpu.VMEM((C * NF_ROWS * W,), jnp.float32),
        pltpu.VMEM((C * NF_ROWS * W,), jnp.float32),
        pltpu.VMEM((NF_ROWS * WPAD,), jnp.float32),
        pltpu.VMEM((NF_ROWS * W,), jnp.float32),
        pltpu.VMEM((NROWS * W,), jnp.float32),
    ],
)
def _sc_saliency(img_hbm, out_hbm, img_v, aa, bb, nf, hs, acc):
    wid = lax.axis_index("s") * 2 + lax.axis_index("c")
    r0 = wid * NROWS

    zeros16 = jnp.zeros((16,), jnp.float32)

    def _zero(ref, n):
        def zb(i, carry):
            ref[pl.ds(i * 16, 16)] = zeros16
            return carry
        lax.fori_loop(0, n // 16, zb, None)

    _zero(img_v, C * IMG_ROWS * WPAD)
    _zero(nf, NF_ROWS * WPAD)
    _zero(acc, NROWS * W)

    for ci in range(C):
        for s in range(IMG_ROWS):
            g = r0 - 3 + s

            @pl.when(jnp.logical_and(g >= 0, g < H))
            def _copy_row(ci=ci, s=s, g=g):
                pltpu.sync_copy(
                    img_hbm.at[pl.ds((ci * H + g) * W, W)],
                    img_v.at[pl.ds((ci * IMG_ROWS + s) * WPAD + IOFF, W)])

    def p1(i, carry):
        r = i // NV
        w0 = (i % NV) * 16
        grow = r0 - 2 + r
        means = []
        ivss = []
        normsq = jnp.full((16,), 108.0, jnp.float32)
        for ci in range(C):
            sm = zeros16
            sq = zeros16
            for dr in (-1, 0, 1):
                for dw in (-1, 0, 1):
                    off = (ci * IMG_ROWS + r + 1 + dr) * WPAD + IOFF + dw + w0
                    p = img_v[pl.ds(off, 16)]
                    sm = sm + p
                    sq = sq + p * p
            mean = sm * (1.0 / 9.0)
            var = jnp.maximum(sq * (1.0 / 9.0) - mean * mean, 0.0)
            ivs = _rsqrt(var + 1e-5)
            normsq = normsq + 9.0 * var * (ivs * ivs)
            means.append(mean)
            ivss.append(ivs)
        invn = _rsqrt(normsq)
        rv = jnp.where(jnp.logical_and(grow >= 0, grow < H), 1.0, 0.0)
        sinvn = invn * rv
        for ci in range(C):
            base = (ci * NF_ROWS + r) * W + w0
            aa[pl.ds(base, 16)] = ivss[ci] * sinvn
            bb[pl.ds(base, 16)] = (2.0 - means[ci] * ivss[ci]) * sinvn
        return carry

    lax.fori_loop(0, NF_ROWS * NV, p1, None)

    def p2(c, carry):
        ci = c // 9
        j = c % 9
        dr = j // 3 - 1
        dw = j % 3 - 1

        def p2a(i, carry):
            r = i // NV
            w0 = (i % NV) * 16
            off = (ci * IMG_ROWS + r + 1 + dr) * WPAD + IOFF + dw + w0
            p = img_v[pl.ds(off, 16)]
            base = (ci * NF_ROWS + r) * W + w0
            a = aa[pl.ds(base, 16)]
            b = bb[pl.ds(base, 16)]
            nf[pl.ds(r * WPAD + IOFF + w0, 16)] = p * a + b
            return carry

        lax.fori_loop(0, NF_ROWS * NV, p2a, None)

        def p2b(i, carry):
            r = i // NV
            w0 = (i % NV) * 16
            s = nf[pl.ds(r * WPAD + IOFF - 2 + w0, 16)]
            for d in (-1, 0, 1, 2):
                s = s + nf[pl.ds(r * WPAD + IOFF + d + w0, 16)]
            hs[pl.ds(r * W + w0, 16)] = s
            return carry

        lax.fori_loop(0, NF_ROWS * NV, p2b, None)

        def p2c(i, carry):
            rr = i // NV
            w0 = (i % NV) * 16
            s5 = hs[pl.ds(rr * W + w0, 16)]
            for d in range(1, 5):
                s5 = s5 + hs[pl.ds((rr + d) * W + w0, 16)]
            ob = rr * W + w0
            acc[pl.ds(ob, 16)] = (
                acc[pl.ds(ob, 16)]
                + nf[pl.ds((rr + 2) * WPAD + IOFF + w0, 16)] * s5)
            return carry

        lax.fori_loop(0, NROWS * NV, p2c, None)
        return carry

    lax.fori_loop(0, C * 9, p2, None)

    iot = lax.iota(jnp.int32, 16)

    def p3(i, carry):
        rr = i // NV
        w0 = (i % NV) * 16
        g = r0 + rr
        rin = jnp.minimum(g + 2, H - 1) - jnp.maximum(g - 2, 0) + 1
        wv = iot + w0
        cin = jnp.minimum(wv + 2, W - 1) - jnp.maximum(wv - 2, 0) + 1
        cnt = (rin * cin - 1).astype(jnp.float32)
        ob = rr * W + w0
        av = acc[pl.ds(ob, 16)]
        acc[pl.ds(ob, 16)] = 0.5 - 0.5 * (av - 1.0) / cnt
        return carry

    lax.fori_loop(0, NROWS * NV, p3, None)

    pltpu.sync_copy(acc, out_hbm.at[pl.ds(r0 * W, NROWS * W)])


def kernel(image):
    img_flat = image.reshape(C * H * W)
    out = _sc_saliency(img_flat)
    return out.reshape(1, H, W)


# speedup vs baseline: 111.0174x; 1.6408x over previous
import functools

import jax
import jax.numpy as jnp
from jax import lax
from jax.experimental import pallas as pl
from jax.experimental.pallas import tpu as pltpu
from jax.experimental.pallas import tpu_sc as plsc

H = 224
W = 224
T = H * W
C = 3
WPAD = 240
IOFF = 8
NROWS = 7
NF_ROWS = 11
IMG_ROWS = 13
NV = W // 16
PLANE = NF_ROWS * WPAD
IPLANE = IMG_ROWS * WPAD


def _floop(lo, hi, init, body):
    return lax.fori_loop(lo, hi, body, init)


def _rsqrt(x):
    i = plsc.bitcast(x, jnp.int32)
    i = jnp.int32(0x5F3759DF) - (i >> 1)
    y = plsc.bitcast(i, jnp.float32)
    for _ in range(3):
        y = y * (1.5 - 0.5 * x * y * y)
    return y


@functools.partial(
    pl.kernel,
    out_type=jax.ShapeDtypeStruct((T,), jnp.float32),
    mesh=plsc.VectorSubcoreMesh(core_axis_name="c", subcore_axis_name="s"),
    compiler_params=pltpu.CompilerParams(needs_layout_passes=False),
    scratch_types=[
        pltpu.VMEM((C * IPLANE,), jnp.float32),
        pltpu.VMEM((C * PLANE,), jnp.float32),
        pltpu.VMEM((C * PLANE,), jnp.float32),
        pltpu.VMEM((C * 9 * PLANE,), jnp.float32),
        pltpu.VMEM((NROWS * W,), jnp.float32),
    ],
)
def _sc_saliency(img_hbm, out_hbm, img_v, aa, bb, nf, outb):
    wid = lax.axis_index("s") * 2 + lax.axis_index("c")
    r0 = wid * NROWS

    zeros16 = jnp.zeros((16,), jnp.float32)

    @functools.partial(_floop, 0, C * IPLANE // 16, jnp.int32(0))
    def _z1(i, off):
        img_v[pl.ds(off, 16)] = zeros16
        return off + 16

    @functools.partial(_floop, 0, C * 9 * NF_ROWS, jnp.int32(0))
    def _z2(i, off):
        nf[pl.ds(off, 16)] = zeros16
        nf[pl.ds(off + 224, 16)] = zeros16
        return off + WPAD

    for ci in range(C):
        for s in range(IMG_ROWS):
            g = r0 - 3 + s

            @pl.when(jnp.logical_and(g >= 0, g < H))
            def _copy_row(ci=ci, s=s, g=g):
                pltpu.sync_copy(
                    img_hbm.at[pl.ds((ci * H + g) * W, W)],
                    img_v.at[pl.ds(ci * IPLANE + s * WPAD + IOFF, W)])

    def _advance(bi, bab, vcnt):
        last = vcnt == NV - 1
        bump = jnp.where(last, 32, 16)
        return bi + bump, bab + bump, jnp.where(last, 0, vcnt + 1)

    @functools.partial(_floop, 0, NF_ROWS * NV,
                       (jnp.int32(IOFF), jnp.int32(IOFF),
                        jnp.int32(0), r0 - 2))
    def _p1(i, carry):
        bi, bab, vcnt, grow = carry
        means = []
        ivss = []
        normsq = jnp.full((16,), 108.0, jnp.float32)
        for ci in range(C):
            sm = zeros16
            sq = zeros16
            for a in (0, 1, 2):
                for b in (-1, 0, 1):
                    p = img_v[pl.ds(bi + ci * IPLANE + a * WPAD + b, 16)]
                    sm = sm + p
                    sq = sq + p * p
            mean = sm * (1.0 / 9.0)
            var = jnp.maximum(sq * (1.0 / 9.0) - mean * mean, 0.0)
            ivs = _rsqrt(var + 1e-5)
            normsq = normsq + 9.0 * var * (ivs * ivs)
            means.append(mean)
            ivss.append(ivs)
        invn = _rsqrt(normsq)
        rv = jnp.where(jnp.logical_and(grow >= 0, grow < H), 1.0, 0.0)
        sinvn = invn * rv
        for ci in range(C):
            aa[pl.ds(bab + ci * PLANE, 16)] = ivss[ci] * sinvn
            bb[pl.ds(bab + ci * PLANE, 16)] = (2.0 - means[ci] * ivss[ci]) * sinvn
        nbi, nbab, nvcnt = _advance(bi, bab, vcnt)
        return (nbi, nbab, nvcnt,
                jnp.where(vcnt == NV - 1, grow + 1, grow))

    for ci in range(C):
        @functools.partial(_floop, 0, NF_ROWS * NV,
                           (jnp.int32(IOFF), jnp.int32(IOFF),
                            jnp.int32(0)))
        def _p2a(i, carry, ci=ci):
            bi, bab, vcnt = carry
            a = aa[pl.ds(bab + ci * PLANE, 16)]
            b = bb[pl.ds(bab + ci * PLANE, 16)]
            for j in range(9):
                dr = j // 3
                dw = j % 3 - 1
                p = img_v[pl.ds(bi + ci * IPLANE + dr * WPAD + dw, 16)]
                nf[pl.ds(bab + (ci * 9 + j) * PLANE, 16)] = p * a + b
            return _advance(bi, bab, vcnt)

    iot = lax.iota(jnp.int32, 16)

    def _strip(v, carry):
        sb, ob = carry

        def _chan(c, ch_carry):
            accs = list(ch_carry[:NROWS])
            pb = ch_carry[NROWS]
            centers = []
            hs = []
            for r in range(NF_ROWS):
                c0 = nf[pl.ds(pb + r * WPAD, 16)]
                s = c0 + nf[pl.ds(pb + r * WPAD - 2, 16)]
                s = s + nf[pl.ds(pb + r * WPAD - 1, 16)]
                s = s + nf[pl.ds(pb + r * WPAD + 1, 16)]
                s = s + nf[pl.ds(pb + r * WPAD + 2, 16)]
                centers.append(c0)
                hs.append(s)
            s5 = hs[0] + hs[1] + hs[2] + hs[3] + hs[4]
            for rr in range(NROWS):
                if rr > 0:
                    s5 = s5 + hs[rr + 4] - hs[rr - 1]
                accs[rr] = accs[rr] + centers[rr + 2] * s5
            return tuple(accs) + (pb + PLANE,)

        init = tuple(zeros16 for _ in range(NROWS)) + (sb,)
        res = lax.fori_loop(0, C * 9, _chan, init)

        wv = iot + (sb - IOFF)
        cin = jnp.minimum(wv + 2, W - 1) - jnp.maximum(wv - 2, 0) + 1
        for rr in range(NROWS):
            g = r0 + rr
            rin = jnp.minimum(g + 2, H - 1) - jnp.maximum(g - 2, 0) + 1
            cnt = (rin * cin - 1).astype(jnp.float32)
            outb[pl.ds(ob + rr * W, 16)] = 0.5 - 0.5 * (res[rr] - 1.0) / cnt
        return (sb + 16, ob + 16)

    lax.fori_loop(0, NV, _strip, (jnp.int32(IOFF), jnp.int32(0)))

    pltpu.sync_copy(outb, out_hbm.at[pl.ds(r0 * W, NROWS * W)])


def kernel(image):
    img_flat = image.reshape(C * H * W)
    out = _sc_saliency(img_flat)
    return out.reshape(1, H, W)


# speedup vs baseline: 141.3282x; 1.2730x over previous
import functools

import jax
import jax.numpy as jnp
from jax import lax
from jax.experimental import pallas as pl
from jax.experimental.pallas import tpu as pltpu
from jax.experimental.pallas import tpu_sc as plsc

H = 224
W = 224
T = H * W
C = 3
WPAD = 240
IOFF = 8
NROWS = 7
NF_ROWS = 11
IMG_ROWS = 13
NV = W // 16
PLANE = NF_ROWS * WPAD
IPLANE = IMG_ROWS * WPAD


def _floop(lo, hi, init, body):
    return lax.fori_loop(lo, hi, body, init)


def _rsqrt(x):
    i = plsc.bitcast(x, jnp.int32)
    i = jnp.int32(0x5F3759DF) - (i >> 1)
    y = plsc.bitcast(i, jnp.float32)
    for _ in range(2):
        y = y * (1.5 - 0.5 * x * y * y)
    return y


@functools.partial(
    pl.kernel,
    out_type=jax.ShapeDtypeStruct((T,), jnp.float32),
    mesh=plsc.VectorSubcoreMesh(core_axis_name="c", subcore_axis_name="s"),
    compiler_params=pltpu.CompilerParams(needs_layout_passes=False),
    scratch_types=[
        pltpu.VMEM((C * IPLANE,), jnp.float32),
        pltpu.VMEM((C * PLANE,), jnp.float32),
        pltpu.VMEM((C * PLANE,), jnp.float32),
        pltpu.VMEM((C * 9 * PLANE,), jnp.float32),
        pltpu.VMEM((NROWS * W,), jnp.float32),
        pltpu.SemaphoreType.DMA,
    ],
)
def _sc_saliency(img_hbm, out_hbm, img_v, aa, bb, nf, outb, dsem):
    wid = lax.axis_index("s") * 2 + lax.axis_index("c")
    r0 = wid * NROWS

    zeros16 = jnp.zeros((16,), jnp.float32)

    @functools.partial(_floop, 0, C * IPLANE // 16, jnp.int32(0))
    def _z1(i, off):
        img_v[pl.ds(off, 16)] = zeros16
        return off + 16

    @functools.partial(_floop, 0, C * 9 * NF_ROWS, jnp.int32(0))
    def _z2(i, off):
        nf[pl.ds(off, 16)] = zeros16
        nf[pl.ds(off + 224, 16)] = zeros16
        return off + WPAD

    for ci in range(C):
        for s in range(IMG_ROWS):
            g = r0 - 3 + s

            @pl.when(jnp.logical_and(g >= 0, g < H))
            def _copy_row(ci=ci, s=s, g=g):
                pltpu.async_copy(
                    img_hbm.at[pl.ds((ci * H + g) * W, W)],
                    img_v.at[pl.ds(ci * IPLANE + s * WPAD + IOFF, W)],
                    dsem)

    ndma = C * (jnp.minimum(r0 + IMG_ROWS - 3, H) - jnp.maximum(r0 - 3, 0))
    _drain = pltpu.make_async_copy(
        img_hbm.at[pl.ds(0, W)], img_v.at[pl.ds(IOFF, W)], dsem)

    def _dr(i, carry):
        _drain.wait()
        return carry

    lax.fori_loop(0, ndma, _dr, None)

    def _advance(bi, bab, vcnt):
        last = vcnt == NV - 1
        bump = jnp.where(last, 32, 16)
        return bi + bump, bab + bump, jnp.where(last, 0, vcnt + 1)

    @functools.partial(_floop, 0, NF_ROWS * NV,
                       (jnp.int32(IOFF), jnp.int32(IOFF),
                        jnp.int32(0), r0 - 2))
    def _p1(i, carry):
        bi, bab, vcnt, grow = carry
        means = []
        ivss = []
        normsq = jnp.full((16,), 108.0, jnp.float32)
        for ci in range(C):
            sm = zeros16
            sq = zeros16
            for a in (0, 1, 2):
                for b in (-1, 0, 1):
                    p = img_v[pl.ds(bi + ci * IPLANE + a * WPAD + b, 16)]
                    sm = sm + p
                    sq = sq + p * p
            mean = sm * (1.0 / 9.0)
            var = jnp.maximum(sq * (1.0 / 9.0) - mean * mean, 0.0)
            ivs = _rsqrt(var + 1e-5)
            normsq = normsq + 9.0 * var * (ivs * ivs)
            means.append(mean)
            ivss.append(ivs)
        invn = _rsqrt(normsq)
        rv = jnp.where(jnp.logical_and(grow >= 0, grow < H), 1.0, 0.0)
        sinvn = invn * rv
        for ci in range(C):
            aa[pl.ds(bab + ci * PLANE, 16)] = ivss[ci] * sinvn
            bb[pl.ds(bab + ci * PLANE, 16)] = (2.0 - means[ci] * ivss[ci]) * sinvn
        nbi, nbab, nvcnt = _advance(bi, bab, vcnt)
        return (nbi, nbab, nvcnt,
                jnp.where(vcnt == NV - 1, grow + 1, grow))

    for ci in range(C):
        @functools.partial(_floop, 0, NF_ROWS * NV,
                           (jnp.int32(IOFF), jnp.int32(IOFF),
                            jnp.int32(0)))
        def _p2a(i, carry, ci=ci):
            bi, bab, vcnt = carry
            a = aa[pl.ds(bab + ci * PLANE, 16)]
            b = bb[pl.ds(bab + ci * PLANE, 16)]
            for j in range(9):
                dr = j // 3
                dw = j % 3 - 1
                p = img_v[pl.ds(bi + ci * IPLANE + dr * WPAD + dw, 16)]
                nf[pl.ds(bab + (ci * 9 + j) * PLANE, 16)] = p * a + b
            return _advance(bi, bab, vcnt)

    iot = lax.iota(jnp.int32, 16)

    def _strip(v, carry):
        sb, ob = carry

        def _chan(c, ch_carry):
            accs = list(ch_carry[:NROWS])
            pb = ch_carry[NROWS]
            centers = []
            hs = []
            for r in range(NF_ROWS):
                c0 = nf[pl.ds(pb + r * WPAD, 16)]
                s = c0 + nf[pl.ds(pb + r * WPAD - 2, 16)]
                s = s + nf[pl.ds(pb + r * WPAD - 1, 16)]
                s = s + nf[pl.ds(pb + r * WPAD + 1, 16)]
                s = s + nf[pl.ds(pb + r * WPAD + 2, 16)]
                centers.append(c0)
                hs.append(s)
            s5 = hs[0] + hs[1] + hs[2] + hs[3] + hs[4]
            for rr in range(NROWS):
                if rr > 0:
                    s5 = s5 + hs[rr + 4] - hs[rr - 1]
                accs[rr] = accs[rr] + centers[rr + 2] * s5
            return tuple(accs) + (pb + PLANE,)

        init = tuple(zeros16 for _ in range(NROWS)) + (sb,)
        res = lax.fori_loop(0, C * 9, _chan, init)

        wv = iot + (sb - IOFF)
        cin = jnp.minimum(wv + 2, W - 1) - jnp.maximum(wv - 2, 0) + 1
        for rr in range(NROWS):
            g = r0 + rr
            rin = jnp.minimum(g + 2, H - 1) - jnp.maximum(g - 2, 0) + 1
            cnt = (rin * cin - 1).astype(jnp.float32)
            outb[pl.ds(ob + rr * W, 16)] = 0.5 - 0.5 * (res[rr] - 1.0) / cnt
        return (sb + 16, ob + 16)

    lax.fori_loop(0, NV, _strip, (jnp.int32(IOFF), jnp.int32(0)))

    pltpu.sync_copy(outb, out_hbm.at[pl.ds(r0 * W, NROWS * W)])


def kernel(image):
    img_flat = image.reshape(C * H * W)
    out = _sc_saliency(img_flat)
    return out.reshape(1, H, W)
